# baseline (device time: 24982 ns/iter reference)
import jax
import jax.numpy as jnp
from jax import lax
from jax.experimental import pallas as pl
from jax.experimental.pallas import tpu as pltpu

N_DEV = 4
E_PER = 4
N_TOK = 1024
N_EXP = 16
D_IN = 256
D_OUT = 512
CHUNK = N_TOK // N_DEV
HALVES = 4
SUB = CHUNK // HALVES


def kernel(x, router_W, route_idx, expert_W, shared_W):
    def body(x_ref, rw_ref, idx_ref, ew_ref, sw_ref, out_ref,
             xs_ref, sb_ref, red_ref, ag_send_ref, rs_buf, ag_buf,
             rs_send_sems, rs_recv_sems, ag_send_sems, ag_recv_sems):
        my = lax.axis_index("i")

        barrier_sem = pltpu.get_barrier_semaphore()
        for s in range(1, N_DEV):
            peer = lax.rem(my + s, N_DEV)
            pl.semaphore_signal(
                barrier_sem, inc=1,
                device_id=(peer,), device_id_type=pl.DeviceIdType.MESH,
            )

        x32 = x_ref[:, :]
        scores = jnp.dot(x32, rw_ref[:, :], preferred_element_type=jnp.float32)
        s_max = jnp.max(scores, axis=-1, keepdims=True)
        e_s = jnp.exp(scores - s_max)
        probs = e_s / jnp.sum(e_s, axis=-1, keepdims=True)
        idx2 = idx_ref[:, :]
        eiota = lax.broadcasted_iota(jnp.int32, (N_TOK, N_EXP), 1)
        p_routed = jnp.sum(
            jnp.where(eiota == idx2, probs, 0.0), axis=-1, keepdims=True
        )

        xbf = x32.astype(jnp.bfloat16)
        xq = (x32 * 0.25).astype(jnp.bfloat16)
        w_stack = jnp.concatenate(
            [sw_ref[:, :].astype(jnp.bfloat16)]
            + [ew_ref[j].astype(jnp.bfloat16) for j in range(E_PER)],
            axis=0,
        )
        scales = []
        for j in range(E_PER):
            e_glob = my * E_PER + j
            scales.append(
                jnp.where(idx2 == e_glob, p_routed, 0.0).astype(jnp.bfloat16)
            )

        xs_ref[:, :] = jnp.concatenate(
            [xq] + [xbf * s for s in scales], axis=1
        )

        def partial_rows(lo, n):
            return jnp.dot(
                xs_ref[lo:lo + n, :], w_stack,
                preferred_element_type=jnp.float32,
            )

        pl.semaphore_wait(barrier_sem, N_DEV - 1)

        for h in range(HALVES):
            for c in range(N_DEV):
                lo = c * CHUNK + h * SUB
                part = partial_rows(lo, SUB)

                @pl.when(my == c)
                def _(h=h, part=part):
                    red_ref[h * SUB:(h + 1) * SUB, :] = part

                @pl.when(my != c)
                def _(h=h, c=c, part=part):
                    sb_ref[h, c] = part.astype(jnp.bfloat16)
                    rdma = pltpu.make_async_remote_copy(
                        src_ref=sb_ref.at[h, c],
                        dst_ref=rs_buf.at[h, my],
                        send_sem=rs_send_sems.at[h, c],
                        recv_sem=rs_recv_sems.at[h, my],
                        device_id=(c,),
                        device_id_type=pl.DeviceIdType.MESH,
                    )
                    rdma.start()

        for h in range(HALVES):
            for k in range(N_DEV):
                @pl.when(my != k)
                def _(h=h, k=k):
                    recv = pltpu.make_async_remote_copy(
                        src_ref=sb_ref.at[h, k],
                        dst_ref=rs_buf.at[h, k],
                        send_sem=rs_send_sems.at[h, k],
                        recv_sem=rs_recv_sems.at[h, k],
                        device_id=(k,),
                        device_id_type=pl.DeviceIdType.MESH,
                    )
                    recv.wait_recv()
                    red_ref[h * SUB:(h + 1) * SUB, :] += rs_buf[h, k].astype(
                        jnp.float32
                    )
            ag_send_ref[h] = red_ref[h * SUB:(h + 1) * SUB, :].astype(
                jnp.bfloat16
            )
            for c in range(N_DEV):
                @pl.when(my != c)
                def _(h=h, c=c):
                    rdma = pltpu.make_async_remote_copy(
                        src_ref=ag_send_ref.at[h],
                        dst_ref=ag_buf.at[h, my],
                        send_sem=ag_send_sems.at[h, c],
                        recv_sem=ag_recv_sems.at[h, my],
                        device_id=(c,),
                        device_id_type=pl.DeviceIdType.MESH,
                    )
                    rdma.start()

        out_ref[pl.ds(my * CHUNK, CHUNK), :] = red_ref[:, :]

        for h in range(HALVES):
            for k in range(N_DEV):
                @pl.when(my != k)
                def _(h=h, k=k):
                    recv = pltpu.make_async_remote_copy(
                        src_ref=ag_send_ref.at[h],
                        dst_ref=ag_buf.at[h, k],
                        send_sem=ag_send_sems.at[h, k],
                        recv_sem=ag_recv_sems.at[h, k],
                        device_id=(k,),
                        device_id_type=pl.DeviceIdType.MESH,
                    )
                    recv.wait_recv()
                    lo = k * CHUNK + h * SUB
                    out_ref[lo:lo + SUB, :] = ag_buf[h, k].astype(jnp.float32)

        for h in range(HALVES):
            for c in range(N_DEV):
                @pl.when(my != c)
                def _(h=h, c=c):
                    send = pltpu.make_async_remote_copy(
                        src_ref=sb_ref.at[h, c],
                        dst_ref=rs_buf.at[h, my],
                        send_sem=rs_send_sems.at[h, c],
                        recv_sem=rs_recv_sems.at[h, my],
                        device_id=(c,),
                        device_id_type=pl.DeviceIdType.MESH,
                    )
                    send.wait_send()
                    send2 = pltpu.make_async_remote_copy(
                        src_ref=ag_send_ref.at[h],
                        dst_ref=ag_buf.at[h, my],
                        send_sem=ag_send_sems.at[h, c],
                        recv_sem=ag_recv_sems.at[h, my],
                        device_id=(c,),
                        device_id_type=pl.DeviceIdType.MESH,
                    )
                    send2.wait_send()

    return pl.pallas_call(
        body,
        out_shape=jax.ShapeDtypeStruct((N_TOK, D_OUT), jnp.float32),
        in_specs=[pl.BlockSpec(memory_space=pltpu.VMEM)] * 5,
        out_specs=pl.BlockSpec(memory_space=pltpu.VMEM),
        scratch_shapes=[
            pltpu.VMEM((N_TOK, (E_PER + 1) * D_IN), jnp.bfloat16),
            pltpu.VMEM((HALVES, N_DEV, SUB, D_OUT), jnp.bfloat16),
            pltpu.VMEM((CHUNK, D_OUT), jnp.float32),
            pltpu.VMEM((HALVES, SUB, D_OUT), jnp.bfloat16),
            pltpu.VMEM((HALVES, N_DEV, SUB, D_OUT), jnp.bfloat16),
            pltpu.VMEM((HALVES, N_DEV, SUB, D_OUT), jnp.bfloat16),
            pltpu.SemaphoreType.DMA((HALVES, N_DEV)),
            pltpu.SemaphoreType.DMA((HALVES, N_DEV)),
            pltpu.SemaphoreType.DMA((HALVES, N_DEV)),
            pltpu.SemaphoreType.DMA((HALVES, N_DEV)),
        ],
        compiler_params=pltpu.CompilerParams(collective_id=0),
    )(x, router_W, route_idx, expert_W, shared_W)
